# baseline (speedup 1.0000x reference)
import functools

import jax
import jax.numpy as jnp
from jax import lax
from jax.experimental import pallas as pl
from jax.experimental.pallas import tpu as pltpu
from jax.experimental.pallas import tpu_sc as plsc

NUM_STEPS = 64
NUM_AGENTS = 500000
_CH = 12800
_NCK = NUM_AGENTS // _CH
_PER_W = 10


def _make_sc_copy():
    mesh = plsc.VectorSubcoreMesh(core_axis_name="c", subcore_axis_name="s")

    @functools.partial(
        pl.kernel, mesh=mesh,
        out_type=jax.ShapeDtypeStruct((NUM_STEPS, NUM_AGENTS), jnp.float32),
        scratch_types=[
            pltpu.VMEM((8, _CH), jnp.float32),
        ],
    )
    def sc_copy(iq_hbm, out_hbm, buf):
        wid = lax.axis_index("s") * 2 + lax.axis_index("c")
        g = wid // 4
        row0 = pl.multiple_of(g * 8, 8)
        for k in range(_PER_W):
            c = (wid % 4) * _PER_W + k

            @pl.when(c < _NCK)
            def _():
                off = pl.multiple_of(c * _CH, 128)
                pltpu.sync_copy(
                    iq_hbm.at[pl.ds(row0, 8), pl.ds(off, _CH)], buf)
                pltpu.sync_copy(
                    buf, out_hbm.at[pl.ds(row0, 8), pl.ds(off, _CH)])

    return sc_copy


@jax.jit
def kernel(is_quarantined, quarantine_start_date, quarantine_start_prob,
           quarantine_break_prob, t):
    return _make_sc_copy()(is_quarantined)


# speedup vs baseline: 1.1682x; 1.1682x over previous
import jax
import jax.numpy as jnp
from jax.experimental import pallas as pl
from jax.experimental.pallas import tpu as pltpu

NUM_STEPS = 64
NUM_AGENTS = 500000
QUARANTINE_DAYS = 10.0
_BC = 32768


def _threefry2x32(k0, k1, x1_in):
    ks0 = k0
    ks1 = k1
    ks2 = k0 ^ k1 ^ jnp.uint32(0x1BD11BDA)
    ks = (ks0, ks1, ks2)
    x0 = jnp.zeros_like(x1_in) + ks0
    x1 = x1_in + ks1
    rotations = ((13, 15, 26, 6), (17, 29, 16, 24))
    for i in range(5):
        for r in rotations[i % 2]:
            x0 = x0 + x1
            x1 = (x1 << jnp.uint32(r)) | (x1 >> jnp.uint32(32 - r))
            x1 = x1 ^ x0
        x0 = x0 + ks[(i + 1) % 3]
        x1 = x1 + ks[(i + 2) % 3] + jnp.uint32(i + 1)
    return x0 ^ x1


def _bits_to_unit(bits):
    f = jax.lax.bitcast_convert_type(
        (bits >> jnp.uint32(9)) | jnp.uint32(0x3F800000), jnp.float32
    ) - jnp.float32(1.0)
    minv = jnp.float32(1e-6)
    maxv = jnp.float32(1.0 - 1e-6)
    return jnp.maximum(minv, f * (maxv - minv) + minv)


def _body(kd_ref, probs_ref, t_ref, iq_ref, qsd_ref, out_ref):
    i = pl.program_id(0)
    tt = t_ref[0]
    bsub = _BC // 8
    a = jax.lax.broadcasted_iota(jnp.int32, (8, bsub), 0)
    b = jax.lax.broadcasted_iota(jnp.int32, (8, bsub), 1)
    col = (i * _BC + a * bsub + b).astype(jnp.uint32)
    bits1 = _threefry2x32(kd_ref[0], kd_ref[1], col)
    bits2 = _threefry2x32(kd_ref[2], kd_ref[3], col)
    u1 = _bits_to_unit(bits1)
    u2 = _bits_to_unit(bits2)
    p1 = jnp.clip(probs_ref[0], jnp.float32(1e-6), jnp.float32(1.0 - 1e-6))
    p2 = jnp.clip(probs_ref[1], jnp.float32(1e-6), jnp.float32(1.0 - 1e-6))
    one = jnp.float32(1.0)
    s = (u1 > one - p1).astype(jnp.float32)
    brk = (u2 > one - p2).astype(jnp.float32)

    t_f = tt.astype(jnp.float32)
    qsd8 = qsd_ref[...].reshape(8, bsub)
    end = (t_f >= qsd8 + jnp.float32(QUARANTINE_DAYS)).astype(jnp.float32)
    r1e = jnp.where(end > jnp.float32(0.5), s, one)
    v1 = r1e * (one - r1e * brk)
    v0 = s * (one - s * brk)
    v0b = v0.reshape(1, _BC)
    mb = (v1 - v0).reshape(1, _BC)

    out_ref[...] = iq_ref[...]
    xrow = iq_ref[pl.ds(tt, 1), :]
    out_ref[pl.ds(tt, 1), :] = v0b + mb * xrow


@jax.jit
def kernel(is_quarantined, quarantine_start_date, quarantine_start_prob,
           quarantine_break_prob, t):
    num_steps, n = is_quarantined.shape
    key = jax.random.fold_in(jax.random.key(1), t)
    k1, k2 = jax.random.split(key)
    kd = jnp.concatenate(
        [jax.random.key_data(k1), jax.random.key_data(k2)]
    ).astype(jnp.uint32)
    probs = jnp.stack(
        [quarantine_start_prob[0], quarantine_break_prob[0]]
    ).astype(jnp.float32)
    t32 = jnp.asarray(t, jnp.int32).reshape(1)
    qsd = quarantine_start_date.astype(jnp.float32).reshape(1, n)

    grid = pl.cdiv(n, _BC)
    out = pl.pallas_call(
        _body,
        grid=(grid,),
        in_specs=[
            pl.BlockSpec(memory_space=pltpu.SMEM),
            pl.BlockSpec(memory_space=pltpu.SMEM),
            pl.BlockSpec(memory_space=pltpu.SMEM),
            pl.BlockSpec((num_steps, _BC), lambda i: (0, i)),
            pl.BlockSpec((1, _BC), lambda i: (0, i)),
        ],
        out_specs=pl.BlockSpec((num_steps, _BC), lambda i: (0, i)),
        out_shape=jax.ShapeDtypeStruct((num_steps, n), jnp.float32),
    )(kd, probs, t32, is_quarantined, qsd)
    return out
